# Initial kernel scaffold; baseline (speedup 1.0000x reference)
import jax
import jax.numpy as jnp
from jax.experimental import pallas as pl


def kernel(f_s, f_t, idx, contrast_idx, W_s, b_s, W_t, b_t, memory_v1, memory_v2):
    raise NotImplementedError("write your pallas kernel here")


# speedup vs baseline: 1.5765x; 1.5765x over previous
import functools
import math

import jax
import jax.numpy as jnp
from jax import lax
from jax.experimental import pallas as pl
from jax.experimental.pallas import tpu as pltpu
from jax.experimental.pallas import tpu_sc as plsc

_EPS = 1e-07
_BATCH = 1024
_FEAT = 128
_NDATA = 1000000
_K1 = 512
_T = 0.07

_NC = 2
_NS = 16
_NW = _NC * _NS
_BPW = _BATCH // _NW
_CH = 128
_NCH = _K1 // _CH
_L = 16


def _embed_body(x_ref, w_ref, b_ref, o_ref):
    x = x_ref[...]
    w = w_ref[...]
    y = lax.dot_general(x, w, (((1,), (1,)), ((), ())),
                        preferred_element_type=jnp.float32)
    y = y + b_ref[...]
    norm = jnp.sqrt(jnp.sum(y * y, axis=1, keepdims=True))
    o_ref[...] = y / norm


def _embed(x, w, b):
    bsz, d = x.shape
    bb = 256
    return pl.pallas_call(
        _embed_body,
        grid=(bsz // bb,),
        in_specs=[
            pl.BlockSpec((bb, d), lambda i: (i, 0)),
            pl.BlockSpec((_FEAT, d), lambda i: (0, 0)),
            pl.BlockSpec((1, _FEAT), lambda i: (0, 0)),
        ],
        out_specs=pl.BlockSpec((bb, _FEAT), lambda i: (i, 0)),
        out_shape=jax.ShapeDtypeStruct((bsz, _FEAT), jnp.float32),
    )(x, w, b.reshape(1, _FEAT))


def _lane_total(acc, lane):
    for s in (8, 4, 2, 1):
        acc = acc + acc.at[lane ^ s].get(mode="promise_in_bounds")
    return acc


def _sc_dots_body(mem1, mem2, cidx, v1, v2, s1o, s2o,
                  idx_v, rows_v, v1_v, v2_v, s1_v, s2_v, sem):
    wid = lax.axis_index("s") * _NC + lax.axis_index("c")
    base = wid * _BPW
    lane = lax.iota(jnp.int32, _L)

    pltpu.sync_copy(v1.at[pl.ds(base, _BPW)], v1_v)
    pltpu.sync_copy(v2.at[pl.ds(base, _BPW)], v2_v)

    def b_loop(bl, carry):
        b = base + bl
        pltpu.sync_copy(cidx.at[b], idx_v)
        v1c = [v1_v[bl, pl.ds(g * _L, _L)] for g in range(_FEAT // _L)]
        v2c = [v2_v[bl, pl.ds(g * _L, _L)] for g in range(_FEAT // _L)]

        def do_bank(mem, vc, s_v):
            def chunk(c, carry2):
                pltpu.async_copy(mem.at[idx_v.at[pl.ds(c * _CH, _CH)]],
                                 rows_v, sem).wait()

                def grp(jj, carry3):
                    res = jnp.zeros((_L,), jnp.float32)
                    for t in range(_L):
                        j = jj * _L + t
                        acc = rows_v[j, pl.ds(0, _L)] * vc[0]
                        for g in range(1, _FEAT // _L):
                            acc = acc + rows_v[j, pl.ds(g * _L, _L)] * vc[g]
                        acc = _lane_total(acc, lane)
                        res = jnp.where(lane == t, acc, res)
                    s_v[bl, pl.ds(c * _CH + jj * _L, _L)] = res
                    return carry3

                return lax.fori_loop(0, _CH // _L, grp, carry2)

            return lax.fori_loop(0, _NCH, chunk, 0)

        do_bank(mem2, v1c, s1_v)
        do_bank(mem1, v2c, s2_v)
        return carry

    lax.fori_loop(0, _BPW, b_loop, 0)
    pltpu.sync_copy(s1_v, s1o.at[pl.ds(base, _BPW)])
    pltpu.sync_copy(s2_v, s2o.at[pl.ds(base, _BPW)])


_sc_dots = functools.partial(
    pl.kernel,
    out_type=(jax.ShapeDtypeStruct((_BATCH, _K1), jnp.float32),
              jax.ShapeDtypeStruct((_BATCH, _K1), jnp.float32)),
    mesh=plsc.VectorSubcoreMesh(core_axis_name="c", subcore_axis_name="s"),
    scratch_types=[
        pltpu.VMEM((_K1,), jnp.int32),
        pltpu.VMEM((_CH, _FEAT), jnp.float32),
        pltpu.VMEM((_BPW, _FEAT), jnp.float32),
        pltpu.VMEM((_BPW, _FEAT), jnp.float32),
        pltpu.VMEM((_BPW, _K1), jnp.float32),
        pltpu.VMEM((_BPW, _K1), jnp.float32),
        pltpu.SemaphoreType.DMA,
    ],
)(_sc_dots_body)


def _loss_body(s1_ref, s2_ref, o_ref):
    m = float(_K1 - 1)
    pn = 1.0 / float(_NDATA)
    mpn = m * pn

    def side(s_ref):
        e = jnp.exp(s_ref[...] * (1.0 / _T))
        z = jnp.mean(e) * float(_NDATA)
        p = e / z
        col0 = p[:, 0:1]
        log_d1 = jnp.log(col0 / (col0 + mpn + _EPS))
        log_d0_all = jnp.log(mpn / (p + mpn + _EPS))
        log_d0_col0 = jnp.log(mpn / (col0 + mpn + _EPS))
        return -(jnp.sum(log_d1) + jnp.sum(log_d0_all)
                 - jnp.sum(log_d0_col0)) / float(_BATCH)

    o_ref[0, 0] = side(s1_ref) + side(s2_ref)


def _loss(s1, s2):
    return pl.pallas_call(
        _loss_body,
        in_specs=[
            pl.BlockSpec((_BATCH, _K1), lambda: (0, 0)),
            pl.BlockSpec((_BATCH, _K1), lambda: (0, 0)),
        ],
        out_specs=pl.BlockSpec(memory_space=pltpu.MemorySpace.SMEM),
        out_shape=jax.ShapeDtypeStruct((1, 1), jnp.float32),
    )(s1, s2)


def kernel(f_s, f_t, idx, contrast_idx, W_s, b_s, W_t, b_t,
           memory_v1, memory_v2):
    del idx
    v1 = _embed(f_s, W_s, b_s)
    v2 = _embed(f_t, W_t, b_t)
    s1, s2 = _sc_dots(memory_v1, memory_v2, contrast_idx, v1, v2)
    return _loss(s1, s2).reshape((1,))


# speedup vs baseline: 2.5067x; 1.5900x over previous
import functools
import math

import jax
import jax.numpy as jnp
from jax import lax
from jax.experimental import pallas as pl
from jax.experimental.pallas import tpu as pltpu
from jax.experimental.pallas import tpu_sc as plsc

_EPS = 1e-07
_BATCH = 1024
_FEAT = 128
_NDATA = 1000000
_K1 = 512
_T = 0.07

_NC = 2
_NS = 16
_NW = _NC * _NS
_BPW = _BATCH // _NW
_CH = 128
_NCH = _K1 // _CH
_L = 16


def _embed_body(x_ref, w_ref, b_ref, o_ref):
    x = x_ref[...]
    w = w_ref[...]
    y = lax.dot_general(x, w, (((1,), (1,)), ((), ())),
                        preferred_element_type=jnp.float32)
    y = y + b_ref[...]
    norm = jnp.sqrt(jnp.sum(y * y, axis=1, keepdims=True))
    o_ref[...] = y / norm


def _embed(x, w, b):
    bsz, d = x.shape
    bb = 256
    return pl.pallas_call(
        _embed_body,
        grid=(bsz // bb,),
        in_specs=[
            pl.BlockSpec((bb, d), lambda i: (i, 0)),
            pl.BlockSpec((_FEAT, d), lambda i: (0, 0)),
            pl.BlockSpec((1, _FEAT), lambda i: (0, 0)),
        ],
        out_specs=pl.BlockSpec((bb, _FEAT), lambda i: (i, 0)),
        out_shape=jax.ShapeDtypeStruct((bsz, _FEAT), jnp.float32),
    )(x, w, b.reshape(1, _FEAT))


def _lane_total(acc, lane):
    for s in (8, 4, 2, 1):
        acc = acc + acc.at[lane ^ s].get(mode="promise_in_bounds")
    return acc


def _sc_dots_body(mem1, mem2, cidx, v1, v2, s1o, s2o,
                  idx_v, rows_v, v1_v, v2_v, s1_v, s2_v, sem0, sem1):
    wid = lax.axis_index("s") * _NC + lax.axis_index("c")
    base = wid * _BPW
    lane = lax.iota(jnp.int32, _L)
    sems = (sem0, sem1)
    banks = (mem2, mem1)

    pltpu.sync_copy(v1.at[pl.ds(base, _BPW)], v1_v)
    pltpu.sync_copy(v2.at[pl.ds(base, _BPW)], v2_v)
    pltpu.sync_copy(cidx.at[pl.ds(base, _BPW)], idx_v)

    def gather(bl, bank, c, buf):
        return pltpu.async_copy(
            banks[bank].at[idx_v.at[bl, pl.ds(c * _CH, _CH)]],
            rows_v.at[buf], sems[buf])

    gather(0, 0, 0, 0)

    def b_loop(bl, carry):
        v1c = [v1_v[bl, pl.ds(g * _L, _L)] for g in range(_FEAT // _L)]
        v2c = [v2_v[bl, pl.ds(g * _L, _L)] for g in range(_FEAT // _L)]
        for seq in range(2 * _NCH):
            bank, c, buf = seq % 2, seq // 2, seq % 2
            if seq + 1 < 2 * _NCH:
                gather(bl, (seq + 1) % 2, (seq + 1) // 2, (seq + 1) % 2)
            else:
                gather(jnp.minimum(bl + 1, _BPW - 1), 0, 0, 0)
            pltpu.make_async_copy(
                banks[bank].at[idx_v.at[bl, pl.ds(c * _CH, _CH)]],
                rows_v.at[buf], sems[buf]).wait()
            vc = v1c if bank == 0 else v2c
            s_v = s1_v if bank == 0 else s2_v

            def grp(jj, carry3):
                res = jnp.zeros((_L,), jnp.float32)
                for t in range(_L):
                    j = jj * _L + t
                    acc = rows_v[buf, j, pl.ds(0, _L)] * vc[0]
                    for g in range(1, _FEAT // _L):
                        acc = acc + rows_v[buf, j, pl.ds(g * _L, _L)] * vc[g]
                    acc = _lane_total(acc, lane)
                    res = jnp.where(lane == t, acc, res)
                s_v[bl, pl.ds(c * _CH + jj * _L, _L)] = res
                return carry3

            lax.fori_loop(0, _CH // _L, grp, 0)
        return carry

    lax.fori_loop(0, _BPW, b_loop, 0)
    pltpu.make_async_copy(
        banks[0].at[idx_v.at[_BPW - 1, pl.ds(0, _CH)]],
        rows_v.at[0], sems[0]).wait()
    pltpu.sync_copy(s1_v, s1o.at[pl.ds(base, _BPW)])
    pltpu.sync_copy(s2_v, s2o.at[pl.ds(base, _BPW)])


_sc_dots = functools.partial(
    pl.kernel,
    out_type=(jax.ShapeDtypeStruct((_BATCH, _K1), jnp.float32),
              jax.ShapeDtypeStruct((_BATCH, _K1), jnp.float32)),
    mesh=plsc.VectorSubcoreMesh(core_axis_name="c", subcore_axis_name="s"),
    scratch_types=[
        pltpu.VMEM((_BPW, _K1), jnp.int32),
        pltpu.VMEM((2, _CH, _FEAT), jnp.float32),
        pltpu.VMEM((_BPW, _FEAT), jnp.float32),
        pltpu.VMEM((_BPW, _FEAT), jnp.float32),
        pltpu.VMEM((_BPW, _K1), jnp.float32),
        pltpu.VMEM((_BPW, _K1), jnp.float32),
        pltpu.SemaphoreType.DMA,
        pltpu.SemaphoreType.DMA,
    ],
)(_sc_dots_body)


def _loss_body(s1_ref, s2_ref, o_ref):
    m = float(_K1 - 1)
    pn = 1.0 / float(_NDATA)
    mpn = m * pn

    def side(s_ref):
        e = jnp.exp(s_ref[...] * (1.0 / _T))
        z = jnp.mean(e) * float(_NDATA)
        p = e / z
        col0 = p[:, 0:1]
        log_d1 = jnp.log(col0 / (col0 + mpn + _EPS))
        log_d0_all = jnp.log(mpn / (p + mpn + _EPS))
        log_d0_col0 = jnp.log(mpn / (col0 + mpn + _EPS))
        return -(jnp.sum(log_d1) + jnp.sum(log_d0_all)
                 - jnp.sum(log_d0_col0)) / float(_BATCH)

    o_ref[0, 0] = side(s1_ref) + side(s2_ref)


def _loss(s1, s2):
    return pl.pallas_call(
        _loss_body,
        in_specs=[
            pl.BlockSpec((_BATCH, _K1), lambda: (0, 0)),
            pl.BlockSpec((_BATCH, _K1), lambda: (0, 0)),
        ],
        out_specs=pl.BlockSpec(memory_space=pltpu.MemorySpace.SMEM),
        out_shape=jax.ShapeDtypeStruct((1, 1), jnp.float32),
    )(s1, s2)


def kernel(f_s, f_t, idx, contrast_idx, W_s, b_s, W_t, b_t,
           memory_v1, memory_v2):
    del idx
    v1 = _embed(f_s, W_s, b_s)
    v2 = _embed(f_t, W_t, b_t)
    s1, s2 = _sc_dots(memory_v1, memory_v2, contrast_idx, v1, v2)
    return _loss(s1, s2).reshape((1,))


# speedup vs baseline: 2.8196x; 1.1248x over previous
import functools
import math

import jax
import jax.numpy as jnp
from jax import lax
from jax.experimental import pallas as pl
from jax.experimental.pallas import tpu as pltpu
from jax.experimental.pallas import tpu_sc as plsc

_EPS = 1e-07
_BATCH = 1024
_FEAT = 128
_NDATA = 1000000
_K1 = 512
_T = 0.07

_NC = 2
_NS = 16
_NW = _NC * _NS
_BPW = _BATCH // _NW
_CH = 128
_DEPTH = 4
_NCH = _K1 // _CH
_L = 16


def _embed_body(x_ref, w_ref, b_ref, o_ref):
    x = x_ref[...]
    w = w_ref[...]
    y = lax.dot_general(x, w, (((1,), (1,)), ((), ())),
                        preferred_element_type=jnp.float32)
    y = y + b_ref[...]
    norm = jnp.sqrt(jnp.sum(y * y, axis=1, keepdims=True))
    o_ref[...] = y / norm


def _embed(x, w, b):
    bsz, d = x.shape
    bb = 256
    return pl.pallas_call(
        _embed_body,
        grid=(bsz // bb,),
        in_specs=[
            pl.BlockSpec((bb, d), lambda i: (i, 0)),
            pl.BlockSpec((_FEAT, d), lambda i: (0, 0)),
            pl.BlockSpec((1, _FEAT), lambda i: (0, 0)),
        ],
        out_specs=pl.BlockSpec((bb, _FEAT), lambda i: (i, 0)),
        out_shape=jax.ShapeDtypeStruct((bsz, _FEAT), jnp.float32),
    )(x, w, b.reshape(1, _FEAT))


def _lane_total(acc, lane):
    for s in (8, 4, 2, 1):
        acc = acc + acc.at[lane ^ s].get(mode="promise_in_bounds")
    return acc


def _sc_dots_body(mem1, mem2, cidx, v1, v2, s1o, s2o,
                  idx_v, rows_v, v1_v, v2_v, s1_v, s2_v,
                  sem0, sem1, sem2, sem3):
    wid = lax.axis_index("s") * _NC + lax.axis_index("c")
    base = wid * _BPW
    lane = lax.iota(jnp.int32, _L)
    sems = (sem0, sem1, sem2, sem3)
    banks = (mem2, mem1)
    nseq = 2 * _NCH

    pltpu.sync_copy(v1.at[pl.ds(base, _BPW)], v1_v)
    pltpu.sync_copy(v2.at[pl.ds(base, _BPW)], v2_v)
    pltpu.sync_copy(cidx.at[pl.ds(base, _BPW)], idx_v)

    def gather(bl, seq, buf):
        return pltpu.async_copy(
            banks[seq % 2].at[idx_v.at[bl, pl.ds((seq // 2) * _CH, _CH)]],
            rows_v.at[buf], sems[buf])

    for s in range(_DEPTH - 1):
        gather(0, s, s % _DEPTH)

    def b_loop(bl, carry):
        v1c = [v1_v[bl, pl.ds(g * _L, _L)] for g in range(_FEAT // _L)]
        v2c = [v2_v[bl, pl.ds(g * _L, _L)] for g in range(_FEAT // _L)]
        for seq in range(nseq):
            bank, c, buf = seq % 2, seq // 2, seq % _DEPTH
            fs = seq + _DEPTH - 1
            if fs < nseq:
                gather(bl, fs, fs % _DEPTH)
            else:
                gather(jnp.minimum(bl + 1, _BPW - 1), fs - nseq, fs % _DEPTH)
            pltpu.make_async_copy(
                banks[bank].at[idx_v.at[bl, pl.ds(c * _CH, _CH)]],
                rows_v.at[buf], sems[buf]).wait()
            vc = v1c if bank == 0 else v2c
            s_v = s1_v if bank == 0 else s2_v

            def grp(jj, carry3):
                res = jnp.zeros((_L,), jnp.float32)
                for t in range(_L):
                    j = jj * _L + t
                    acc = rows_v[buf, j, pl.ds(0, _L)] * vc[0]
                    for g in range(1, _FEAT // _L):
                        acc = acc + rows_v[buf, j, pl.ds(g * _L, _L)] * vc[g]
                    acc = _lane_total(acc, lane)
                    res = jnp.where(lane == t, acc, res)
                s_v[bl, pl.ds(c * _CH + jj * _L, _L)] = res
                return carry3

            lax.fori_loop(0, _CH // _L, grp, 0)
        return carry

    lax.fori_loop(0, _BPW, b_loop, 0)
    for s in range(_DEPTH - 1):
        pltpu.make_async_copy(
            banks[s % 2].at[idx_v.at[_BPW - 1, pl.ds((s // 2) * _CH, _CH)]],
            rows_v.at[s % _DEPTH], sems[s % _DEPTH]).wait()
    pltpu.sync_copy(s1_v, s1o.at[pl.ds(base, _BPW)])
    pltpu.sync_copy(s2_v, s2o.at[pl.ds(base, _BPW)])


_sc_dots = functools.partial(
    pl.kernel,
    out_type=(jax.ShapeDtypeStruct((_BATCH, _K1), jnp.float32),
              jax.ShapeDtypeStruct((_BATCH, _K1), jnp.float32)),
    mesh=plsc.VectorSubcoreMesh(core_axis_name="c", subcore_axis_name="s"),
    scratch_types=[
        pltpu.VMEM((_BPW, _K1), jnp.int32),
        pltpu.VMEM((_DEPTH, _CH, _FEAT), jnp.float32),
        pltpu.VMEM((_BPW, _FEAT), jnp.float32),
        pltpu.VMEM((_BPW, _FEAT), jnp.float32),
        pltpu.VMEM((_BPW, _K1), jnp.float32),
        pltpu.VMEM((_BPW, _K1), jnp.float32),
        pltpu.SemaphoreType.DMA,
        pltpu.SemaphoreType.DMA,
        pltpu.SemaphoreType.DMA,
        pltpu.SemaphoreType.DMA,
    ],
)(_sc_dots_body)


def _loss_body(s1_ref, s2_ref, o_ref):
    m = float(_K1 - 1)
    pn = 1.0 / float(_NDATA)
    mpn = m * pn

    def side(s_ref):
        e = jnp.exp(s_ref[...] * (1.0 / _T))
        z = jnp.mean(e) * float(_NDATA)
        p = e / z
        col0 = p[:, 0:1]
        log_d1 = jnp.log(col0 / (col0 + mpn + _EPS))
        log_d0_all = jnp.log(mpn / (p + mpn + _EPS))
        log_d0_col0 = jnp.log(mpn / (col0 + mpn + _EPS))
        return -(jnp.sum(log_d1) + jnp.sum(log_d0_all)
                 - jnp.sum(log_d0_col0)) / float(_BATCH)

    o_ref[0, 0] = side(s1_ref) + side(s2_ref)


def _loss(s1, s2):
    return pl.pallas_call(
        _loss_body,
        in_specs=[
            pl.BlockSpec((_BATCH, _K1), lambda: (0, 0)),
            pl.BlockSpec((_BATCH, _K1), lambda: (0, 0)),
        ],
        out_specs=pl.BlockSpec(memory_space=pltpu.MemorySpace.SMEM),
        out_shape=jax.ShapeDtypeStruct((1, 1), jnp.float32),
    )(s1, s2)


def kernel(f_s, f_t, idx, contrast_idx, W_s, b_s, W_t, b_t,
           memory_v1, memory_v2):
    del idx
    v1 = _embed(f_s, W_s, b_s)
    v2 = _embed(f_t, W_t, b_t)
    s1, s2 = _sc_dots(memory_v1, memory_v2, contrast_idx, v1, v2)
    return _loss(s1, s2).reshape((1,))
